# baseline (speedup 1.0000x reference)
import functools

import jax
import jax.numpy as jnp
from jax import lax
from jax.experimental import pallas as pl
from jax.experimental.pallas import tpu as pltpu
from jax.experimental.pallas import tpu_sc as plsc

B = 16384
D0 = 32
D1 = 64
DC = D0 + D1
NC = 2
NS = 16
NW = NC * NS
CHUNK = 128
CPW = B // (NW * CHUNK)

_mesh = plsc.VectorSubcoreMesh(core_axis_name="c", subcore_axis_name="s")


@functools.partial(
    pl.kernel,
    mesh=_mesh,
    out_type=[
        jax.ShapeDtypeStruct((B, DC), jnp.float32),
        jax.ShapeDtypeStruct((B, DC), jnp.float32),
    ],
    scratch_types=[
        pltpu.VMEM((CPW, CHUNK), jnp.int32),
        pltpu.VMEM((CPW, CHUNK), jnp.int32),
        pltpu.VMEM((CHUNK, D0), jnp.float32),
        pltpu.VMEM((CHUNK, D1), jnp.float32),
        pltpu.VMEM((CHUNK, D0), jnp.float32),
        pltpu.VMEM((CHUNK, D1), jnp.float32),
        pltpu.SemaphoreType.DMA,
    ],
    compiler_params=pltpu.CompilerParams(use_tc_tiling_on_sc=False),
)
def _mixed_emb(it0, ut0, it1, ut1, iid, uid, item_out, user_out,
               iidx_v, uidx_v, vi0, vi1, vu0, vu1, sem):
    wid = lax.axis_index("s") * NC + lax.axis_index("c")
    row0 = wid * CPW
    pltpu.sync_copy(iid.at[pl.ds(row0, CPW)], iidx_v)
    pltpu.sync_copy(uid.at[pl.ds(row0, CPW)], uidx_v)
    for c in range(CPW):
        base = (row0 + c) * CHUNK
        d0 = pltpu.async_copy(it0.at[iidx_v.at[c]], vi0, sem)
        d1 = pltpu.async_copy(it1.at[iidx_v.at[c]], vi1, sem)
        d2 = pltpu.async_copy(ut0.at[uidx_v.at[c]], vu0, sem)
        d3 = pltpu.async_copy(ut1.at[uidx_v.at[c]], vu1, sem)
        d0.wait()
        d1.wait()
        d2.wait()
        d3.wait()
        pltpu.sync_copy(vi0, item_out.at[pl.ds(base, CHUNK), pl.ds(0, D0)])
        pltpu.sync_copy(vi1, item_out.at[pl.ds(base, CHUNK), pl.ds(D0, D1)])
        pltpu.sync_copy(vu0, user_out.at[pl.ds(base, CHUNK), pl.ds(0, D0)])
        pltpu.sync_copy(vu1, user_out.at[pl.ds(base, CHUNK), pl.ds(D0, D1)])


def kernel(item_table0, user_table0, item_table1, user_table1, item_ids, user_ids):
    iid = item_ids.reshape(B // CHUNK, CHUNK)
    uid = user_ids.reshape(B // CHUNK, CHUNK)
    item_out, user_out = _mixed_emb(
        item_table0, user_table0, item_table1, user_table1, iid, uid)
    return item_out, user_out


# speedup vs baseline: 1.0137x; 1.0137x over previous
import functools

import jax
import jax.numpy as jnp
from jax import lax
from jax.experimental import pallas as pl
from jax.experimental.pallas import tpu as pltpu
from jax.experimental.pallas import tpu_sc as plsc

B = 16384
D0 = 32
D1 = 64
DC = D0 + D1
NC = 2
NS = 16
NW = NC * NS
CHUNK = 128
CPW = B // (NW * CHUNK)
NW_ROWS = CPW * CHUNK

_mesh = plsc.VectorSubcoreMesh(core_axis_name="c", subcore_axis_name="s")


@functools.partial(
    pl.kernel,
    mesh=_mesh,
    out_type=[
        jax.ShapeDtypeStruct((B, DC), jnp.float32),
        jax.ShapeDtypeStruct((B, DC), jnp.float32),
    ],
    scratch_types=[
        pltpu.VMEM((CPW, CHUNK), jnp.int32),
        pltpu.VMEM((CPW, CHUNK), jnp.int32),
        pltpu.VMEM((NW_ROWS, D0), jnp.float32),
        pltpu.VMEM((NW_ROWS, D1), jnp.float32),
        pltpu.VMEM((NW_ROWS, D0), jnp.float32),
        pltpu.VMEM((NW_ROWS, D1), jnp.float32),
        pltpu.SemaphoreType.DMA,
        pltpu.SemaphoreType.DMA,
        pltpu.SemaphoreType.DMA,
        pltpu.SemaphoreType.DMA,
        pltpu.SemaphoreType.DMA,
    ],
    compiler_params=pltpu.CompilerParams(use_tc_tiling_on_sc=False),
)
def _mixed_emb(it0, ut0, it1, ut1, iid, uid, item_out, user_out,
               iidx_v, uidx_v, vi0, vi1, vu0, vu1,
               s_i0, s_i1, s_u0, s_u1, s_w):
    wid = lax.axis_index("s") * NC + lax.axis_index("c")
    row0 = wid * CPW
    base = row0 * CHUNK
    pltpu.sync_copy(iid.at[pl.ds(row0, CPW)], iidx_v)
    pltpu.sync_copy(uid.at[pl.ds(row0, CPW)], uidx_v)
    gathers = []
    for c in range(CPW):
        lo = pl.ds(c * CHUNK, CHUNK)
        gathers.append((
            pltpu.async_copy(it0.at[iidx_v.at[c]], vi0.at[lo], s_i0),
            pltpu.async_copy(it1.at[iidx_v.at[c]], vi1.at[lo], s_i1),
            pltpu.async_copy(ut0.at[uidx_v.at[c]], vu0.at[lo], s_u0),
            pltpu.async_copy(ut1.at[uidx_v.at[c]], vu1.at[lo], s_u1),
        ))
    writes = []
    rows = pl.ds(base, NW_ROWS)
    for t, (buf, out, col) in enumerate((
            (vi0, item_out, pl.ds(0, D0)),
            (vi1, item_out, pl.ds(D0, D1)),
            (vu0, user_out, pl.ds(0, D0)),
            (vu1, user_out, pl.ds(D0, D1)))):
        for c in range(CPW):
            gathers[c][t].wait()
        writes.append(pltpu.async_copy(buf, out.at[rows, col], s_w))
    for w in writes:
        w.wait()


def kernel(item_table0, user_table0, item_table1, user_table1, item_ids, user_ids):
    iid = item_ids.reshape(B // CHUNK, CHUNK)
    uid = user_ids.reshape(B // CHUNK, CHUNK)
    item_out, user_out = _mixed_emb(
        item_table0, user_table0, item_table1, user_table1, iid, uid)
    return item_out, user_out


# speedup vs baseline: 1.0417x; 1.0277x over previous
import functools

import jax
import jax.numpy as jnp
from jax import lax
from jax.experimental import pallas as pl
from jax.experimental.pallas import tpu as pltpu
from jax.experimental.pallas import tpu_sc as plsc

B = 16384
D0 = 32
D1 = 64
DC = D0 + D1
NC = 2
NS = 16
NW = NC * NS
CHUNK = 128
CPW = B // (NW * CHUNK)
NW_ROWS = CPW * CHUNK

_mesh = plsc.VectorSubcoreMesh(core_axis_name="c", subcore_axis_name="s")


@functools.partial(
    pl.kernel,
    mesh=_mesh,
    out_type=jax.ShapeDtypeStruct((B, DC), jnp.float32),
    scratch_types=[
        pltpu.VMEM((NW_ROWS,), jnp.int32),
        pltpu.VMEM((NW_ROWS, D0), jnp.float32),
        pltpu.VMEM((NW_ROWS, D1), jnp.float32),
        pltpu.SemaphoreType.DMA,
        pltpu.SemaphoreType.DMA,
        pltpu.SemaphoreType.DMA,
    ],
    compiler_params=pltpu.CompilerParams(use_tc_tiling_on_sc=False),
)
def _pair_gather(t0, t1, ids, out, idx_v, v0, v1, s_0, s_1, s_w):
    wid = lax.axis_index("s") * NC + lax.axis_index("c")
    base = wid * NW_ROWS
    pltpu.sync_copy(ids.at[pl.ds(base, NW_ROWS)], idx_v)
    gathers = []
    for c in range(CPW):
        isl = pl.ds(c * CHUNK, CHUNK)
        rows = pl.ds(c * CHUNK, CHUNK)
        gathers.append((
            pltpu.async_copy(t0.at[idx_v.at[isl]], v0.at[rows], s_0),
            pltpu.async_copy(t1.at[idx_v.at[isl]], v1.at[rows], s_1),
        ))
    for c in range(CPW):
        gathers[c][0].wait()
    w0 = pltpu.async_copy(v0, out.at[pl.ds(base, NW_ROWS), pl.ds(0, D0)], s_w)
    for c in range(CPW):
        gathers[c][1].wait()
    w1 = pltpu.async_copy(v1, out.at[pl.ds(base, NW_ROWS), pl.ds(D0, D1)], s_w)
    w0.wait()
    w1.wait()


def kernel(item_table0, user_table0, item_table1, user_table1, item_ids, user_ids):
    item_out = _pair_gather(item_table0, item_table1, item_ids)
    user_out = _pair_gather(user_table0, user_table1, user_ids)
    return item_out, user_out
